# baseline (speedup 1.0000x reference)
import functools

import jax
import jax.numpy as jnp
from jax import lax
from jax.experimental import pallas as pl
from jax.experimental.pallas import tpu as pltpu, tpu_sc as plsc

N = 10000
E = 320000
D = 128
NPAD = 10240
RANGE = NPAD // 16
EHALF = E // 2
CHUNK = 2000
GROUPS = CHUNK // 16
NCHUNK = EHALF // CHUNK
SHOT = 64


def _agg_body(xproj, src_hbm, dst_hbm, out_hbm, acc, dstv, srcv, dstq, srcq,
              rows, sem):
    c = lax.axis_index("c")
    s = lax.axis_index("s")
    lo = s * RANGE
    ebase = c * EHALF

    zero16 = jnp.zeros((16,), jnp.float32)

    def zero_row(i, _):
        for u in range(8):
            acc[i, pl.ds(u * 16, 16)] = zero16
        return 0

    lax.fori_loop(0, RANGE + 1, zero_row, 0)

    cols = lax.iota(jnp.int32, 16)

    def chunk_body(ci, _):
        off = ebase + ci * CHUNK
        pltpu.sync_copy(dst_hbm.at[pl.ds(off, CHUNK)], dstv)
        pltpu.sync_copy(src_hbm.at[pl.ds(off, CHUNK)], srcv)

        def scan_body(g, cnt):
            d = dstv[pl.ds(g * 16, 16)]
            sv = srcv[pl.ds(g * 16, 16)]
            dl = d - lo
            m = (dl >= 0) & (dl < RANGE)
            mi = m.astype(jnp.int32)
            pos = cnt + plsc.cumsum(mi) - 1
            plsc.store_scatter(dstq, [pos], dl, mask=m)
            plsc.store_scatter(srcq, [pos], sv, mask=m)
            return cnt + jnp.sum(mi)

        cnt = lax.fori_loop(0, GROUPS, scan_body, 0)

        dummy_d = jnp.full((16,), RANGE, jnp.int32)
        dummy_s = jnp.zeros((16,), jnp.int32)
        for i in range(SHOT // 16):
            dstq[pl.ds(cnt + i * 16, 16)] = dummy_d
            srcq[pl.ds(cnt + i * 16, 16)] = dummy_s

        nshots = (cnt + SHOT - 1) // SHOT

        def shot_body(sh, _):
            qoff = sh * SHOT
            pltpu.async_copy(xproj.at[srcq.at[pl.ds(qoff, SHOT)]], rows,
                             sem).wait()

            def edge_body(j, _):
                dvec = plsc.load_gather(
                    dstq, [jnp.full((16,), qoff + j, jnp.int32)])
                for u in range(8):
                    cu = cols + (u * 16)
                    a = plsc.load_gather(acc, [dvec, cu])
                    g = rows[j, pl.ds(u * 16, 16)]
                    plsc.store_scatter(acc, [dvec, cu], jnp.maximum(a, g))
                return 0

            lax.fori_loop(0, SHOT, edge_body, 0)
            return 0

        lax.fori_loop(0, nshots, shot_body, 0)
        return 0

    lax.fori_loop(0, NCHUNK, chunk_body, 0)

    pltpu.sync_copy(acc.at[pl.ds(0, RANGE)], out_hbm.at[c, pl.ds(lo, RANGE)])


def _segment_max_sc(xproj, src, dst):
    mesh = plsc.VectorSubcoreMesh(core_axis_name="c", subcore_axis_name="s")
    fn = functools.partial(
        pl.kernel,
        mesh=mesh,
        compiler_params=pltpu.CompilerParams(needs_layout_passes=False),
        out_type=jax.ShapeDtypeStruct((2, NPAD, D), jnp.float32),
        scratch_types=[
            pltpu.VMEM((RANGE + 1, D), jnp.float32),
            pltpu.VMEM((CHUNK,), jnp.int32),
            pltpu.VMEM((CHUNK,), jnp.int32),
            pltpu.VMEM((CHUNK + SHOT,), jnp.int32),
            pltpu.VMEM((CHUNK + SHOT,), jnp.int32),
            pltpu.VMEM((SHOT, D), jnp.float32),
            pltpu.SemaphoreType.DMA,
        ],
    )(_agg_body)
    return fn(xproj, src, dst)


def _proj_body(x_ref, w_ref, b_ref, o_ref):
    o_ref[:] = jax.nn.relu(
        jnp.dot(x_ref[:], w_ref[:], preferred_element_type=jnp.float32)
        + b_ref[:])


def _proj(x, w, b):
    m = x.shape[0]
    blk = 1280
    grid = m // blk
    return pl.pallas_call(
        _proj_body,
        grid=(grid,),
        in_specs=[
            pl.BlockSpec((blk, D), lambda i: (i, 0)),
            pl.BlockSpec((D, D), lambda i: (0, 0)),
            pl.BlockSpec((1, D), lambda i: (0, 0)),
        ],
        out_specs=pl.BlockSpec((blk, D), lambda i: (i, 0)),
        out_shape=jax.ShapeDtypeStruct((m, D), jnp.float32),
    )(x, w, b.reshape(1, D))


def _combine_body(p_ref, x_ref, wl_ref, bl_ref, wr_ref, wp_ref, bp_ref,
                  h_ref, xp_ref):
    a = jnp.maximum(p_ref[0], p_ref[1])
    h = jax.nn.relu(
        jnp.dot(a, wl_ref[:], preferred_element_type=jnp.float32)
        + jnp.dot(x_ref[:], wr_ref[:], preferred_element_type=jnp.float32)
        + bl_ref[:])
    h_ref[:] = h
    xp_ref[:] = jax.nn.relu(
        jnp.dot(h, wp_ref[:], preferred_element_type=jnp.float32)
        + bp_ref[:])


def _combine_proj(p, x, wl, bl, wr, wp, bp):
    blk = 1280
    grid = NPAD // blk
    return pl.pallas_call(
        _combine_body,
        grid=(grid,),
        in_specs=[
            pl.BlockSpec((2, blk, D), lambda i: (0, i, 0)),
            pl.BlockSpec((blk, D), lambda i: (i, 0)),
            pl.BlockSpec((D, D), lambda i: (0, 0)),
            pl.BlockSpec((1, D), lambda i: (0, 0)),
            pl.BlockSpec((D, D), lambda i: (0, 0)),
            pl.BlockSpec((D, D), lambda i: (0, 0)),
            pl.BlockSpec((1, D), lambda i: (0, 0)),
        ],
        out_specs=[
            pl.BlockSpec((blk, D), lambda i: (i, 0)),
            pl.BlockSpec((blk, D), lambda i: (i, 0)),
        ],
        out_shape=[
            jax.ShapeDtypeStruct((NPAD, D), jnp.float32),
            jax.ShapeDtypeStruct((NPAD, D), jnp.float32),
        ],
    )(p, x, wl, bl.reshape(1, D), wr, wp, bp.reshape(1, D))


def _final_body(p_ref, x_ref, wl_ref, bl_ref, wr_ref, o_ref):
    a = jnp.maximum(p_ref[0], p_ref[1])
    o_ref[:] = jax.nn.relu(
        jnp.dot(a, wl_ref[:], preferred_element_type=jnp.float32)
        + jnp.dot(x_ref[:], wr_ref[:], preferred_element_type=jnp.float32)
        + bl_ref[:])


def _final(p, x, wl, bl, wr):
    blk = 1280
    grid = NPAD // blk
    return pl.pallas_call(
        _final_body,
        grid=(grid,),
        in_specs=[
            pl.BlockSpec((2, blk, D), lambda i: (0, i, 0)),
            pl.BlockSpec((blk, D), lambda i: (i, 0)),
            pl.BlockSpec((D, D), lambda i: (0, 0)),
            pl.BlockSpec((1, D), lambda i: (0, 0)),
            pl.BlockSpec((D, D), lambda i: (0, 0)),
        ],
        out_specs=pl.BlockSpec((blk, D), lambda i: (i, 0)),
        out_shape=jax.ShapeDtypeStruct((NPAD, D), jnp.float32),
    )(p, x, wl, bl.reshape(1, D), wr)


def kernel(x, edge_index, Wp1, bp1, Wl1, bl1, Wr1, Wp2, bp2, Wl2, bl2, Wr2):
    src = edge_index[0]
    dst = edge_index[1]
    x_pad = jnp.pad(x, ((0, NPAD - N), (0, 0)))

    xp1 = _proj(x_pad, Wp1, bp1)
    p1 = _segment_max_sc(xp1, src, dst)
    h, xp2 = _combine_proj(p1, x_pad, Wl1, bl1, Wr1, Wp2, bp2)
    p2 = _segment_max_sc(xp2, src, dst)
    out = _final(p2, h, Wl2, bl2, Wr2)
    return out[:N]


# speedup vs baseline: 3.1472x; 3.1472x over previous
import functools

import jax
import jax.numpy as jnp
from jax import lax
from jax.experimental import pallas as pl
from jax.experimental.pallas import tpu as pltpu, tpu_sc as plsc

N = 10000
E = 320000
D = 128
NPAD = 10240
RANGE = NPAD // 16
EHALF = E // 2
CHUNK = 2000
GROUPS = CHUNK // 16
NCHUNK = EHALF // CHUNK
SHOT = 64


def _agg_body(xproj, src_hbm, dst_hbm, out_hbm, acc, dstv, srcv, dstq, srcq,
              rows, sem):
    c = lax.axis_index("c")
    s = lax.axis_index("s")
    lo = s * RANGE
    ebase = c * EHALF

    zero16 = jnp.zeros((16,), jnp.float32)

    def zero_row(i, _):
        for u in range(8):
            acc[i, pl.ds(u * 16, 16)] = zero16
        return 0

    lax.fori_loop(0, RANGE + 1, zero_row, 0)

    cols = lax.iota(jnp.int32, 16)

    def chunk_body(ci, _):
        off = ebase + ci * CHUNK
        pltpu.sync_copy(dst_hbm.at[pl.ds(off, CHUNK)], dstv)
        pltpu.sync_copy(src_hbm.at[pl.ds(off, CHUNK)], srcv)

        def scan_body(g, cnt):
            d = dstv[pl.ds(g * 16, 16)]
            sv = srcv[pl.ds(g * 16, 16)]
            dl = d - lo
            m = (dl >= 0) & (dl < RANGE)
            mi = m.astype(jnp.int32)
            pos = cnt + plsc.cumsum(mi) - 1
            plsc.store_scatter(dstq, [pos], dl, mask=m)
            plsc.store_scatter(srcq, [pos], sv, mask=m)
            return cnt + jnp.sum(mi)

        cnt = lax.fori_loop(0, GROUPS, scan_body, 0)

        dummy_d = jnp.full((16,), RANGE, jnp.int32)
        dummy_s = (s * 32 + c * 16) + cols
        for i in range(SHOT // 16):
            dstq[pl.ds(cnt + i * 16, 16)] = dummy_d
            srcq[pl.ds(cnt + i * 16, 16)] = dummy_s

        nshots = (cnt + SHOT - 1) // SHOT

        def shot_body(sh, _):
            qoff = sh * SHOT
            pltpu.async_copy(xproj.at[srcq.at[pl.ds(qoff, SHOT)]], rows,
                             sem).wait()

            def edge_body(j, _):
                dvec = plsc.load_gather(
                    dstq, [jnp.full((16,), qoff + j, jnp.int32)])
                for u in range(8):
                    cu = cols + (u * 16)
                    a = plsc.load_gather(acc, [dvec, cu])
                    g = rows[j, pl.ds(u * 16, 16)]
                    plsc.store_scatter(acc, [dvec, cu], jnp.maximum(a, g))
                return 0

            lax.fori_loop(0, SHOT, edge_body, 0)
            return 0

        lax.fori_loop(0, nshots, shot_body, 0)
        return 0

    lax.fori_loop(0, NCHUNK, chunk_body, 0)

    pltpu.sync_copy(acc.at[pl.ds(0, RANGE)], out_hbm.at[c, pl.ds(lo, RANGE)])


def _segment_max_sc(xproj, src, dst):
    mesh = plsc.VectorSubcoreMesh(core_axis_name="c", subcore_axis_name="s")
    fn = functools.partial(
        pl.kernel,
        mesh=mesh,
        compiler_params=pltpu.CompilerParams(needs_layout_passes=False),
        out_type=jax.ShapeDtypeStruct((2, NPAD, D), jnp.float32),
        scratch_types=[
            pltpu.VMEM((RANGE + 1, D), jnp.float32),
            pltpu.VMEM((CHUNK,), jnp.int32),
            pltpu.VMEM((CHUNK,), jnp.int32),
            pltpu.VMEM((CHUNK + SHOT,), jnp.int32),
            pltpu.VMEM((CHUNK + SHOT,), jnp.int32),
            pltpu.VMEM((SHOT, D), jnp.float32),
            pltpu.SemaphoreType.DMA,
        ],
    )(_agg_body)
    return fn(xproj, src, dst)


def _proj_body(x_ref, w_ref, b_ref, o_ref):
    o_ref[:] = jax.nn.relu(
        jnp.dot(x_ref[:], w_ref[:], preferred_element_type=jnp.float32)
        + b_ref[:])


def _proj(x, w, b):
    m = x.shape[0]
    blk = 1280
    grid = m // blk
    return pl.pallas_call(
        _proj_body,
        grid=(grid,),
        in_specs=[
            pl.BlockSpec((blk, D), lambda i: (i, 0)),
            pl.BlockSpec((D, D), lambda i: (0, 0)),
            pl.BlockSpec((1, D), lambda i: (0, 0)),
        ],
        out_specs=pl.BlockSpec((blk, D), lambda i: (i, 0)),
        out_shape=jax.ShapeDtypeStruct((m, D), jnp.float32),
    )(x, w, b.reshape(1, D))


def _combine_body(p_ref, x_ref, wl_ref, bl_ref, wr_ref, wp_ref, bp_ref,
                  h_ref, xp_ref):
    a = jnp.maximum(p_ref[0], p_ref[1])
    h = jax.nn.relu(
        jnp.dot(a, wl_ref[:], preferred_element_type=jnp.float32)
        + jnp.dot(x_ref[:], wr_ref[:], preferred_element_type=jnp.float32)
        + bl_ref[:])
    h_ref[:] = h
    xp_ref[:] = jax.nn.relu(
        jnp.dot(h, wp_ref[:], preferred_element_type=jnp.float32)
        + bp_ref[:])


def _combine_proj(p, x, wl, bl, wr, wp, bp):
    blk = 1280
    grid = NPAD // blk
    return pl.pallas_call(
        _combine_body,
        grid=(grid,),
        in_specs=[
            pl.BlockSpec((2, blk, D), lambda i: (0, i, 0)),
            pl.BlockSpec((blk, D), lambda i: (i, 0)),
            pl.BlockSpec((D, D), lambda i: (0, 0)),
            pl.BlockSpec((1, D), lambda i: (0, 0)),
            pl.BlockSpec((D, D), lambda i: (0, 0)),
            pl.BlockSpec((D, D), lambda i: (0, 0)),
            pl.BlockSpec((1, D), lambda i: (0, 0)),
        ],
        out_specs=[
            pl.BlockSpec((blk, D), lambda i: (i, 0)),
            pl.BlockSpec((blk, D), lambda i: (i, 0)),
        ],
        out_shape=[
            jax.ShapeDtypeStruct((NPAD, D), jnp.float32),
            jax.ShapeDtypeStruct((NPAD, D), jnp.float32),
        ],
    )(p, x, wl, bl.reshape(1, D), wr, wp, bp.reshape(1, D))


def _final_body(p_ref, x_ref, wl_ref, bl_ref, wr_ref, o_ref):
    a = jnp.maximum(p_ref[0], p_ref[1])
    o_ref[:] = jax.nn.relu(
        jnp.dot(a, wl_ref[:], preferred_element_type=jnp.float32)
        + jnp.dot(x_ref[:], wr_ref[:], preferred_element_type=jnp.float32)
        + bl_ref[:])


def _final(p, x, wl, bl, wr):
    blk = 1280
    grid = NPAD // blk
    return pl.pallas_call(
        _final_body,
        grid=(grid,),
        in_specs=[
            pl.BlockSpec((2, blk, D), lambda i: (0, i, 0)),
            pl.BlockSpec((blk, D), lambda i: (i, 0)),
            pl.BlockSpec((D, D), lambda i: (0, 0)),
            pl.BlockSpec((1, D), lambda i: (0, 0)),
            pl.BlockSpec((D, D), lambda i: (0, 0)),
        ],
        out_specs=pl.BlockSpec((blk, D), lambda i: (i, 0)),
        out_shape=jax.ShapeDtypeStruct((NPAD, D), jnp.float32),
    )(p, x, wl, bl.reshape(1, D), wr)


def kernel(x, edge_index, Wp1, bp1, Wl1, bl1, Wr1, Wp2, bp2, Wl2, bl2, Wr2):
    src = edge_index[0]
    dst = edge_index[1]
    x_pad = jnp.pad(x, ((0, NPAD - N), (0, 0)))

    xp1 = _proj(x_pad, Wp1, bp1)
    p1 = _segment_max_sc(xp1, src, dst)
    h, xp2 = _combine_proj(p1, x_pad, Wl1, bl1, Wr1, Wp2, bp2)
    p2 = _segment_max_sc(xp2, src, dst)
    out = _final(p2, h, Wl2, bl2, Wr2)
    return out[:N]


# speedup vs baseline: 4.4324x; 1.4084x over previous
import functools

import jax
import jax.numpy as jnp
from jax import lax
from jax.experimental import pallas as pl
from jax.experimental.pallas import tpu as pltpu, tpu_sc as plsc

N = 10000
E = 320000
D = 128
NPAD = 10240
RANGE = NPAD // 16
EHALF = E // 2
CHUNK = 2000
GROUPS = CHUNK // 16
NCHUNK = EHALF // CHUNK
SHOT = 64


DRAIN = 4096
QCAP = DRAIN + CHUNK + SHOT
NBUF = 3


def _agg_body(xproj, src_hbm, dst_hbm, out_hbm, acc, dstv, srcv, dstq, srcq,
              rows, sem0, sem1, sem2):
    c = lax.axis_index("c")
    s = lax.axis_index("s")
    lo = s * RANGE
    ebase = c * EHALF
    sems = [sem0, sem1, sem2]

    zero16 = jnp.zeros((16,), jnp.float32)

    def zero_row(i, _):
        for u in range(8):
            acc[i, pl.ds(u * 16, 16)] = zero16
        return 0

    lax.fori_loop(0, RANGE + 1, zero_row, 0)

    cols = lax.iota(jnp.int32, 16)
    dummy_d = jnp.full((16,), RANGE, jnp.int32)
    dummy_s = (s * 32 + c * 16) + cols

    def issue(i, k):
        pltpu.async_copy(xproj.at[srcq.at[pl.ds(i * SHOT, SHOT)]],
                         rows.at[k], sems[k])

    def wait(k):
        pltpu.make_async_copy(xproj.at[pl.ds(0, SHOT)], rows.at[k],
                              sems[k]).wait()

    def accumulate(i, k):
        def edge_body(j, _):
            dvec = plsc.load_gather(
                dstq, [jnp.full((16,), i * SHOT + j, jnp.int32)])
            for u in range(8):
                cu = cols + (u * 16)
                a = plsc.load_gather(acc, [dvec, cu])
                g = rows[k, j, pl.ds(u * 16, 16)]
                plsc.store_scatter(acc, [dvec, cu], jnp.maximum(a, g))
            return 0

        lax.fori_loop(0, SHOT, edge_body, 0)

    def drain(cnt):
        for i in range(SHOT // 16):
            dstq[pl.ds(cnt + i * 16, 16)] = dummy_d
            srcq[pl.ds(cnt + i * 16, 16)] = dummy_s
        nsh = (cnt + SHOT - 1) // SHOT

        for k in range(NBUF):
            @pl.when(k < nsh)
            def _():
                issue(k, k)

        def super_body(g, _):
            for k in range(NBUF):
                i = g * NBUF + k

                @pl.when(i < nsh)
                def _():
                    wait(k)
                    accumulate(i, k)

                    @pl.when(i + NBUF < nsh)
                    def _():
                        issue(i + NBUF, k)
            return 0

        lax.fori_loop(0, (nsh + NBUF - 1) // NBUF, super_body, 0)

    def chunk_body(ci, cnt):
        off = ebase + ci * CHUNK
        pltpu.sync_copy(dst_hbm.at[pl.ds(off, CHUNK)], dstv)
        pltpu.sync_copy(src_hbm.at[pl.ds(off, CHUNK)], srcv)

        def scan_body(g, cnt):
            d = dstv[pl.ds(g * 16, 16)]
            sv = srcv[pl.ds(g * 16, 16)]
            dl = d - lo
            m = (dl >= 0) & (dl < RANGE)
            mi = m.astype(jnp.int32)
            pos = cnt + plsc.cumsum(mi) - 1
            plsc.store_scatter(dstq, [pos], dl, mask=m)
            plsc.store_scatter(srcq, [pos], sv, mask=m)
            return cnt + jnp.sum(mi)

        cnt = lax.fori_loop(0, GROUPS, scan_body, cnt)

        def do_drain(cnt):
            drain(cnt)
            return cnt * 0

        return lax.cond(cnt >= DRAIN, do_drain, lambda cnt: cnt, cnt)

    cnt = lax.fori_loop(0, NCHUNK, chunk_body, 0)

    @pl.when(cnt > 0)
    def _():
        drain(cnt)

    pltpu.sync_copy(acc.at[pl.ds(0, RANGE)], out_hbm.at[c, pl.ds(lo, RANGE)])


def _segment_max_sc(xproj, src, dst):
    mesh = plsc.VectorSubcoreMesh(core_axis_name="c", subcore_axis_name="s")
    fn = functools.partial(
        pl.kernel,
        mesh=mesh,
        compiler_params=pltpu.CompilerParams(needs_layout_passes=False),
        out_type=jax.ShapeDtypeStruct((2, NPAD, D), jnp.float32),
        scratch_types=[
            pltpu.VMEM((RANGE + 1, D), jnp.float32),
            pltpu.VMEM((CHUNK,), jnp.int32),
            pltpu.VMEM((CHUNK,), jnp.int32),
            pltpu.VMEM((QCAP,), jnp.int32),
            pltpu.VMEM((QCAP,), jnp.int32),
            pltpu.VMEM((NBUF, SHOT, D), jnp.float32),
            pltpu.SemaphoreType.DMA,
            pltpu.SemaphoreType.DMA,
            pltpu.SemaphoreType.DMA,
        ],
    )(_agg_body)
    return fn(xproj, src, dst)


def _proj_body(x_ref, w_ref, b_ref, o_ref):
    o_ref[:] = jax.nn.relu(
        jnp.dot(x_ref[:], w_ref[:], preferred_element_type=jnp.float32)
        + b_ref[:])


def _proj(x, w, b):
    m = x.shape[0]
    blk = 1280
    grid = m // blk
    return pl.pallas_call(
        _proj_body,
        grid=(grid,),
        in_specs=[
            pl.BlockSpec((blk, D), lambda i: (i, 0)),
            pl.BlockSpec((D, D), lambda i: (0, 0)),
            pl.BlockSpec((1, D), lambda i: (0, 0)),
        ],
        out_specs=pl.BlockSpec((blk, D), lambda i: (i, 0)),
        out_shape=jax.ShapeDtypeStruct((m, D), jnp.float32),
    )(x, w, b.reshape(1, D))


def _combine_body(p_ref, x_ref, wl_ref, bl_ref, wr_ref, wp_ref, bp_ref,
                  h_ref, xp_ref):
    a = jnp.maximum(p_ref[0], p_ref[1])
    h = jax.nn.relu(
        jnp.dot(a, wl_ref[:], preferred_element_type=jnp.float32)
        + jnp.dot(x_ref[:], wr_ref[:], preferred_element_type=jnp.float32)
        + bl_ref[:])
    h_ref[:] = h
    xp_ref[:] = jax.nn.relu(
        jnp.dot(h, wp_ref[:], preferred_element_type=jnp.float32)
        + bp_ref[:])


def _combine_proj(p, x, wl, bl, wr, wp, bp):
    blk = 1280
    grid = NPAD // blk
    return pl.pallas_call(
        _combine_body,
        grid=(grid,),
        in_specs=[
            pl.BlockSpec((2, blk, D), lambda i: (0, i, 0)),
            pl.BlockSpec((blk, D), lambda i: (i, 0)),
            pl.BlockSpec((D, D), lambda i: (0, 0)),
            pl.BlockSpec((1, D), lambda i: (0, 0)),
            pl.BlockSpec((D, D), lambda i: (0, 0)),
            pl.BlockSpec((D, D), lambda i: (0, 0)),
            pl.BlockSpec((1, D), lambda i: (0, 0)),
        ],
        out_specs=[
            pl.BlockSpec((blk, D), lambda i: (i, 0)),
            pl.BlockSpec((blk, D), lambda i: (i, 0)),
        ],
        out_shape=[
            jax.ShapeDtypeStruct((NPAD, D), jnp.float32),
            jax.ShapeDtypeStruct((NPAD, D), jnp.float32),
        ],
    )(p, x, wl, bl.reshape(1, D), wr, wp, bp.reshape(1, D))


def _final_body(p_ref, x_ref, wl_ref, bl_ref, wr_ref, o_ref):
    a = jnp.maximum(p_ref[0], p_ref[1])
    o_ref[:] = jax.nn.relu(
        jnp.dot(a, wl_ref[:], preferred_element_type=jnp.float32)
        + jnp.dot(x_ref[:], wr_ref[:], preferred_element_type=jnp.float32)
        + bl_ref[:])


def _final(p, x, wl, bl, wr):
    blk = 1280
    grid = NPAD // blk
    return pl.pallas_call(
        _final_body,
        grid=(grid,),
        in_specs=[
            pl.BlockSpec((2, blk, D), lambda i: (0, i, 0)),
            pl.BlockSpec((blk, D), lambda i: (i, 0)),
            pl.BlockSpec((D, D), lambda i: (0, 0)),
            pl.BlockSpec((1, D), lambda i: (0, 0)),
            pl.BlockSpec((D, D), lambda i: (0, 0)),
        ],
        out_specs=pl.BlockSpec((blk, D), lambda i: (i, 0)),
        out_shape=jax.ShapeDtypeStruct((NPAD, D), jnp.float32),
    )(p, x, wl, bl.reshape(1, D), wr)


def kernel(x, edge_index, Wp1, bp1, Wl1, bl1, Wr1, Wp2, bp2, Wl2, bl2, Wr2):
    src = edge_index[0]
    dst = edge_index[1]
    x_pad = jnp.pad(x, ((0, NPAD - N), (0, 0)))

    xp1 = _proj(x_pad, Wp1, bp1)
    p1 = _segment_max_sc(xp1, src, dst)
    h, xp2 = _combine_proj(p1, x_pad, Wl1, bl1, Wr1, Wp2, bp2)
    p2 = _segment_max_sc(xp2, src, dst)
    out = _final(p2, h, Wl2, bl2, Wr2)
    return out[:N]
